# Initial kernel scaffold; baseline (speedup 1.0000x reference)
import jax
import jax.numpy as jnp
from jax.experimental import pallas as pl


def kernel(v_tensor, vn_tensor, C, a, b):
    raise NotImplementedError("write your pallas kernel here")


# speedup vs baseline: 2.2563x; 2.2563x over previous
import functools

import jax
import jax.numpy as jnp
from jax import lax
from jax.experimental import pallas as pl
from jax.experimental.pallas import tpu as pltpu
from jax.experimental.pallas import tpu_sc as plsc

B = 4
F = 100000
N = 3 * F
H = 512
HH = H * H
ROWS = 2 * HH
TRASH = 128
ROWSP = ROWS + TRASH

W = 128
NSUB = 16
NWT = 152
NW = NWT * NSUB
NP = NW * W
SLICE = ROWS // NSUB
ZCHUNK = 4096

STAGE_T = 2048
POST_T = 2048

ROLE_A = (0, 1, 2)
ROLE_B = (3, 4)


def _prep_body(pts_ref, nrm_ref, a_ref, b_ref, idx_ref, zc_ref,
               nx_ref, ny_ref, nz_ref):
    w = pl.program_id(1)
    pos = w * STAGE_T + lax.broadcasted_iota(jnp.int32, (1, STAGE_T), 1)
    valid = pos < N

    x = pts_ref[0, 0:1, :]
    y = pts_ref[0, 1:2, :]
    z = pts_ref[0, 2:3, :]
    a0, a1, a2 = a_ref[0], a_ref[1], a_ref[2]
    b0, b1, b2 = b_ref[0], b_ref[1], b_ref[2]

    px = jnp.clip(jnp.round(x * a0 + b0), 0.0, float(H - 1)).astype(jnp.int32)
    py = jnp.clip(jnp.round(y * a1 + b1), 0.0, float(H - 1)).astype(jnp.int32)
    zv = z * a2 + b2

    nx = nrm_ref[0, 0:1, :]
    ny = nrm_ref[0, 1:2, :]
    nz = nrm_ref[0, 2:3, :]
    w_front = (nz < 0.0).astype(jnp.int32)

    row = w_front * HH + py * H + px
    trash = ROWS + (pos & (TRASH - 1))
    shp = (1, 1, STAGE_T)
    idx_ref[...] = jnp.where(valid, row, trash).reshape(shp)
    zero = jnp.zeros_like(zv)
    zc_ref[...] = jnp.where(valid, zv, zero).reshape(shp)
    nx_ref[...] = jnp.where(valid, nx, zero).reshape(shp)
    ny_ref[...] = jnp.where(valid, ny, zero).reshape(shp)
    nz_ref[...] = jnp.where(valid, nz, zero).reshape(shp)


def _prep(pts, nrm, av, bv):
    nwb = NP // STAGE_T
    grid = (B, nwb)
    last_in = (N - 1) // STAGE_T
    io_spec = pl.BlockSpec((1, 3, STAGE_T),
                           lambda b, w: (b, 0, jnp.minimum(w, last_in)))
    s_spec = pl.BlockSpec(memory_space=pltpu.SMEM)
    out_spec = pl.BlockSpec((1, 1, STAGE_T),
                            lambda b, w: (b * nwb + w, 0, 0))
    f32 = jnp.float32
    return pl.pallas_call(
        _prep_body,
        grid=grid,
        in_specs=[io_spec, io_spec, s_spec, s_spec],
        out_specs=[out_spec] * 5,
        out_shape=[jax.ShapeDtypeStruct((B * nwb, 1, STAGE_T), jnp.int32)]
        + [jax.ShapeDtypeStruct((B * nwb, 1, STAGE_T), f32)] * 4,
    )(pts, nrm, av, bv)


def _sc_body(idx_h, zc_h, nx_h, ny_h, nz_h, out_h,
             acc0, acc1, acc2, zbuf, onesb, idxw, valw0, valw1):
    c = lax.axis_index("c")
    s = lax.axis_index("s")
    accs = (acc0, acc1, acc2)
    srcs = (None, zc_h, nx_h, ny_h, nz_h)

    def init_zbuf(i, carry):
        zbuf[pl.ds(i * 16, 16)] = jnp.zeros((16,), jnp.float32)
        return carry

    lax.fori_loop(0, ZCHUNK // 16, init_zbuf, 0)
    for i in range(W // 16):
        onesb[pl.ds(i * 16, 16)] = jnp.ones((16,), jnp.float32)

    def do_batch(b, chans):
        nch = len(chans)

        def zero_step(i, carry):
            for k in range(nch):
                pltpu.sync_copy(zbuf, accs[k].at[pl.ds(s * SLICE + i * ZCHUNK,
                                                       ZCHUNK)])
            return carry

        lax.fori_loop(0, SLICE // ZCHUNK, zero_step, 0)
        plsc.subcore_barrier()

        w0 = s * NWT

        def stage_and_scatter(wbase, nw):
            pltpu.sync_copy(idx_h.at[b, pl.ds(wbase, nw)],
                            idxw.at[pl.ds(0, nw)])
            nxt = [valw0, valw1]
            vbufs = []
            for k in range(nch):
                if srcs[chans[k]] is None:
                    vbufs.append(None)
                else:
                    vbufs.append(nxt.pop(0))
            for k in range(nch):
                if vbufs[k] is not None:
                    pltpu.sync_copy(srcs[chans[k]].at[b, pl.ds(wbase, nw)],
                                    vbufs[k].at[pl.ds(0, nw)])

            def scat(j, carry):
                irow = idxw.at[j]
                for k in range(nch):
                    vsrc = onesb if vbufs[k] is None else vbufs[k].at[j]
                    pltpu.sync_copy(vsrc, accs[k].at[irow], add=True)
                return carry

            lax.fori_loop(0, nw, scat, 0)

        def round_body(r, carry):
            stage_and_scatter(w0 + r * 16, 16)
            return carry

        lax.fori_loop(0, NWT // 16, round_body, 0)
        stage_and_scatter(w0 + (NWT // 16) * 16, NWT % 16)

        plsc.subcore_barrier()
        for k in range(nch):
            base = (chans[k] * B + b) * ROWS
            pltpu.sync_copy(accs[k].at[pl.ds(s * SLICE, SLICE)],
                            out_h.at[pl.ds(base + s * SLICE, SLICE)])

    for b in range(B):
        a_core = b % 2

        @pl.when(c == a_core)
        def _():
            do_batch(b, ROLE_A)

        @pl.when(c != a_core)
        def _():
            do_batch(b, ROLE_B)


def _scatter(idx3, zc3, nx3, ny3, nz3):
    f32 = jnp.float32
    return pl.kernel(
        _sc_body,
        out_type=jax.ShapeDtypeStruct((5 * B * ROWS,), f32),
        mesh=plsc.VectorSubcoreMesh(core_axis_name="c", subcore_axis_name="s"),
        scratch_types=[
            pltpu.VMEM_SHARED((ROWSP,), f32),
            pltpu.VMEM_SHARED((ROWSP,), f32),
            pltpu.VMEM_SHARED((ROWSP,), f32),
            pltpu.VMEM((ZCHUNK,), f32),
            pltpu.VMEM((W,), f32),
            pltpu.VMEM((16, W), jnp.int32),
            pltpu.VMEM((16, W), f32),
            pltpu.VMEM((16, W), f32),
        ],
    )(idx3, zc3, nx3, ny3, nz3)


def _post_body(bk0, bk1, bk2, bk3, bk4, fr0, fr1, fr2, fr3, fr4,
               res_ref, dF_ref, dB_ref, fx_ref, fy_ref, fz_ref,
               bx_ref, by_ref, bz_ref):
    shp = (1, 1, POST_T)

    def rd(ref):
        return ref[0, 0, :].reshape(shp)

    cB = rd(bk0)
    cF = rd(fr0)
    res_ref[...] = (cF + cB).astype(jnp.int32)
    dF_ref[...] = rd(fr1) / jnp.maximum(cF, 1.0)
    dB_ref[...] = rd(bk1) / jnp.maximum(cB, 1.0)

    fx, fy, fz = rd(fr2), rd(fr3), rd(fr4)
    tF = jnp.sqrt(fx * fx + fy * fy + fz * fz)
    tF = jnp.where(tF == 0.0, 1.0, tF)
    fx_ref[...] = fx / tF
    fy_ref[...] = fy / tF
    fz_ref[...] = fz / tF

    bx, by, bz = rd(bk2), rd(bk3), rd(bk4)
    tB = jnp.sqrt(bx * bx + by * by + bz * bz)
    tB = jnp.where(tB == 0.0, 1.0, tB)
    bx_ref[...] = bx / tB
    by_ref[...] = by / tB
    bz_ref[...] = bz / tB


def _post(acc):
    nt = HH // POST_T
    nrt = ROWS // POST_T
    grid = (B, nt)
    acc3 = acc.reshape(5 * B * nrt, 1, POST_T)

    def bk(ci):
        return pl.BlockSpec((1, 1, POST_T),
                            lambda b, t, ci=ci: ((ci * B + b) * nrt + t, 0, 0))

    def fr(ci):
        return pl.BlockSpec(
            (1, 1, POST_T),
            lambda b, t, ci=ci: ((ci * B + b) * nrt + nt + t, 0, 0))

    out_spec = pl.BlockSpec((1, 1, POST_T), lambda b, t: (b * nt + t, 0, 0))
    f32 = jnp.float32
    return pl.pallas_call(
        _post_body,
        grid=grid,
        in_specs=[bk(0), bk(1), bk(2), bk(3), bk(4),
                  fr(0), fr(1), fr(2), fr(3), fr(4)],
        out_specs=[out_spec] * 9,
        out_shape=[jax.ShapeDtypeStruct((B * nt, 1, POST_T), jnp.int32)]
        + [jax.ShapeDtypeStruct((B * nt, 1, POST_T), f32)] * 8,
    )(acc3, acc3, acc3, acc3, acc3, acc3, acc3, acc3, acc3, acc3)


def kernel(v_tensor, vn_tensor, C, a, b):
    del C
    pts = jnp.swapaxes(v_tensor.reshape(B, N, 3), 1, 2)
    nrm = jnp.swapaxes(vn_tensor.reshape(B, N, 3), 1, 2)
    av = a.reshape(3)
    bv = b.reshape(3)

    idx, zc, nx, ny, nz = _prep(pts, nrm, av, bv)
    acc = _scatter(idx.reshape(B, NW, W), zc.reshape(B, NW, W),
                   nx.reshape(B, NW, W), ny.reshape(B, NW, W),
                   nz.reshape(B, NW, W)).reshape(5, B, ROWS)
    res, dF, dB, fx, fy, fz, bx, by, bz = _post(acc)

    cnt = B * HH
    res = res.reshape(cnt)
    dF = dF.reshape(cnt)
    dB = dB.reshape(cnt)
    nF = jnp.stack([fx, fy, fz], axis=-1).reshape(cnt, 3)
    nB = jnp.stack([bx, by, bz], axis=-1).reshape(cnt, 3)
    return (res, dF, dB, nF, nB)


# speedup vs baseline: 21.2859x; 9.4340x over previous
import functools

import jax
import jax.numpy as jnp
from jax import lax
from jax.experimental import pallas as pl
from jax.experimental.pallas import tpu as pltpu
from jax.experimental.pallas import tpu_sc as plsc

B = 4
F = 100000
N = 3 * F
H = 512
HH = H * H
ROWS = 2 * HH
TRASH = 128
ROWSP = ROWS + TRASH

W = 128
NSUB = 16
FP = 114688
NP = 3 * FP
NW = NP // W
NWT = NW // NSUB
SLICE = ROWS // NSUB
ZCHUNK = 4096

FT = 2048
POST_T = 2048

ROLE_A = (0, 1, 2)
ROLE_B = (3, 4)


def _prep_body(vt_ref, nt_ref, a_ref, b_ref, idx_ref, zc_ref,
               nx_ref, ny_ref, nz_ref):
    t = pl.program_id(0)
    bb = pl.program_id(1)
    v = pl.program_id(2)
    posf = t * FT + lax.broadcasted_iota(jnp.int32, (1, FT), 1)
    valid = posf < F

    def row(ref, c):
        r = (v * 3 + c) * B + bb
        return ref[pl.ds(r, 1), :]

    x = row(vt_ref, 0)
    y = row(vt_ref, 1)
    z = row(vt_ref, 2)
    a0, a1, a2 = a_ref[0], a_ref[1], a_ref[2]
    b0, b1, b2 = b_ref[0], b_ref[1], b_ref[2]

    px = jnp.clip(jnp.round(x * a0 + b0), 0.0, float(H - 1)).astype(jnp.int32)
    py = jnp.clip(jnp.round(y * a1 + b1), 0.0, float(H - 1)).astype(jnp.int32)
    zv = z * a2 + b2

    nx = row(nt_ref, 0)
    ny = row(nt_ref, 1)
    nz = row(nt_ref, 2)
    w_front = (nz < 0.0).astype(jnp.int32)

    rowi = w_front * HH + py * H + px
    trash = ROWS + (posf & (TRASH - 1))
    shp = (1, FT // W, W)
    idx_ref[...] = jnp.where(valid, rowi, trash).reshape(shp)
    zero = jnp.zeros_like(zv)
    zc_ref[...] = jnp.where(valid, zv, zero).reshape(shp)
    nx_ref[...] = jnp.where(valid, nx, zero).reshape(shp)
    ny_ref[...] = jnp.where(valid, ny, zero).reshape(shp)
    nz_ref[...] = jnp.where(valid, nz, zero).reshape(shp)


def _prep(vt, nt, av, bv):
    fsteps = FP // FT
    wblk = FT // W
    grid = (fsteps, B, 3)
    last_in = (F - 1) // FT
    io_spec = pl.BlockSpec((9 * B, FT),
                           lambda t, b, v: (0, jnp.minimum(t, last_in)))
    s_spec = pl.BlockSpec(memory_space=pltpu.SMEM)
    out_spec = pl.BlockSpec((1, wblk, W),
                            lambda t, b, v: (b, v * fsteps + t, 0))
    f32 = jnp.float32
    return pl.pallas_call(
        _prep_body,
        grid=grid,
        in_specs=[io_spec, io_spec, s_spec, s_spec],
        out_specs=[out_spec] * 5,
        out_shape=[jax.ShapeDtypeStruct((B, NW, W), jnp.int32)]
        + [jax.ShapeDtypeStruct((B, NW, W), f32)] * 4,
    )(vt, nt, av, bv)


def _sc_body(idx_h, zc_h, nx_h, ny_h, nz_h, out_h,
             acc0, acc1, acc2, zbuf, onesb, idxw, valw0, valw1):
    c = lax.axis_index("c")
    s = lax.axis_index("s")
    accs = (acc0, acc1, acc2)
    srcs = (None, zc_h, nx_h, ny_h, nz_h)

    def init_zbuf(i, carry):
        zbuf[pl.ds(i * 16, 16)] = jnp.zeros((16,), jnp.float32)
        return carry

    lax.fori_loop(0, ZCHUNK // 16, init_zbuf, 0)
    for i in range(W // 16):
        onesb[pl.ds(i * 16, 16)] = jnp.ones((16,), jnp.float32)

    def do_batch(b, chans):
        nch = len(chans)

        def zero_step(i, carry):
            for k in range(nch):
                pltpu.sync_copy(zbuf, accs[k].at[pl.ds(s * SLICE + i * ZCHUNK,
                                                       ZCHUNK)])
            return carry

        lax.fori_loop(0, SLICE // ZCHUNK, zero_step, 0)
        plsc.subcore_barrier()

        w0 = s * NWT

        def stage_and_scatter(wbase, nw):
            pltpu.sync_copy(idx_h.at[b, pl.ds(wbase, nw)],
                            idxw.at[pl.ds(0, nw)])
            nxt = [valw0, valw1]
            vbufs = []
            for k in range(nch):
                if srcs[chans[k]] is None:
                    vbufs.append(None)
                else:
                    vbufs.append(nxt.pop(0))
            for k in range(nch):
                if vbufs[k] is not None:
                    pltpu.sync_copy(srcs[chans[k]].at[b, pl.ds(wbase, nw)],
                                    vbufs[k].at[pl.ds(0, nw)])

            def scat(j, carry):
                irow = idxw.at[j]
                for k in range(nch):
                    vsrc = onesb if vbufs[k] is None else vbufs[k].at[j]
                    pltpu.sync_copy(vsrc, accs[k].at[irow], add=True)
                return carry

            lax.fori_loop(0, nw, scat, 0)

        def round_body(r, carry):
            stage_and_scatter(w0 + r * 16, 16)
            return carry

        lax.fori_loop(0, NWT // 16, round_body, 0)
        stage_and_scatter(w0 + (NWT // 16) * 16, NWT % 16)

        plsc.subcore_barrier()
        for k in range(nch):
            base = (chans[k] * B + b) * ROWS
            pltpu.sync_copy(accs[k].at[pl.ds(s * SLICE, SLICE)],
                            out_h.at[pl.ds(base + s * SLICE, SLICE)])

    for b in range(B):
        a_core = b % 2

        @pl.when(c == a_core)
        def _():
            do_batch(b, ROLE_A)

        @pl.when(c != a_core)
        def _():
            do_batch(b, ROLE_B)


def _scatter(idx3, zc3, nx3, ny3, nz3):
    f32 = jnp.float32
    return pl.kernel(
        _sc_body,
        out_type=jax.ShapeDtypeStruct((5 * B * ROWS,), f32),
        mesh=plsc.VectorSubcoreMesh(core_axis_name="c", subcore_axis_name="s"),
        scratch_types=[
            pltpu.VMEM_SHARED((ROWSP,), f32),
            pltpu.VMEM_SHARED((ROWSP,), f32),
            pltpu.VMEM_SHARED((ROWSP,), f32),
            pltpu.VMEM((ZCHUNK,), f32),
            pltpu.VMEM((W,), f32),
            pltpu.VMEM((16, W), jnp.int32),
            pltpu.VMEM((16, W), f32),
            pltpu.VMEM((16, W), f32),
        ],
    )(idx3, zc3, nx3, ny3, nz3)


def _post_body(bk0, bk1, bk2, bk3, bk4, fr0, fr1, fr2, fr3, fr4,
               res_ref, dF_ref, dB_ref, fx_ref, fy_ref, fz_ref,
               bx_ref, by_ref, bz_ref):
    shp = (1, 1, POST_T)

    def rd(ref):
        return ref[0, 0, :].reshape(shp)

    cB = rd(bk0)
    cF = rd(fr0)
    res_ref[...] = (cF + cB).astype(jnp.int32)
    dF_ref[...] = rd(fr1) / jnp.maximum(cF, 1.0)
    dB_ref[...] = rd(bk1) / jnp.maximum(cB, 1.0)

    fx, fy, fz = rd(fr2), rd(fr3), rd(fr4)
    tF = jnp.sqrt(fx * fx + fy * fy + fz * fz)
    tF = jnp.where(tF == 0.0, 1.0, tF)
    fx_ref[...] = fx / tF
    fy_ref[...] = fy / tF
    fz_ref[...] = fz / tF

    bx, by, bz = rd(bk2), rd(bk3), rd(bk4)
    tB = jnp.sqrt(bx * bx + by * by + bz * bz)
    tB = jnp.where(tB == 0.0, 1.0, tB)
    bx_ref[...] = bx / tB
    by_ref[...] = by / tB
    bz_ref[...] = bz / tB


def _post(acc):
    nt = HH // POST_T
    nrt = ROWS // POST_T
    grid = (B, nt)
    acc3 = acc.reshape(5 * B * nrt, 1, POST_T)

    def bk(ci):
        return pl.BlockSpec((1, 1, POST_T),
                            lambda b, t, ci=ci: ((ci * B + b) * nrt + t, 0, 0))

    def fr(ci):
        return pl.BlockSpec(
            (1, 1, POST_T),
            lambda b, t, ci=ci: ((ci * B + b) * nrt + nt + t, 0, 0))

    out_spec = pl.BlockSpec((1, 1, POST_T), lambda b, t: (b * nt + t, 0, 0))
    f32 = jnp.float32
    return pl.pallas_call(
        _post_body,
        grid=grid,
        in_specs=[bk(0), bk(1), bk(2), bk(3), bk(4),
                  fr(0), fr(1), fr(2), fr(3), fr(4)],
        out_specs=[out_spec] * 9,
        out_shape=[jax.ShapeDtypeStruct((B * nt, 1, POST_T), jnp.int32)]
        + [jax.ShapeDtypeStruct((B * nt, 1, POST_T), f32)] * 8,
    )(acc3, acc3, acc3, acc3, acc3, acc3, acc3, acc3, acc3, acc3)


def kernel(v_tensor, vn_tensor, C, a, b):
    del C
    vt = jnp.transpose(v_tensor, (2, 3, 0, 1)).reshape(9 * B, F)
    nt = jnp.transpose(vn_tensor, (2, 3, 0, 1)).reshape(9 * B, F)
    av = a.reshape(3)
    bv = b.reshape(3)

    idx, zc, nx, ny, nz = _prep(vt, nt, av, bv)
    acc = _scatter(idx, zc, nx, ny, nz).reshape(5, B, ROWS)
    res, dF, dB, fx, fy, fz, bx, by, bz = _post(acc)

    cnt = B * HH
    res = res.reshape(cnt)
    dF = dF.reshape(cnt)
    dB = dB.reshape(cnt)
    nF = jnp.stack([fx, fy, fz], axis=-1).reshape(cnt, 3)
    nB = jnp.stack([bx, by, bz], axis=-1).reshape(cnt, 3)
    return (res, dF, dB, nF, nB)


# speedup vs baseline: 22.0037x; 1.0337x over previous
import functools

import jax
import jax.numpy as jnp
from jax import lax
from jax.experimental import pallas as pl
from jax.experimental.pallas import tpu as pltpu
from jax.experimental.pallas import tpu_sc as plsc

B = 4
F = 100000
N = 3 * F
H = 512
HH = H * H
ROWS = 2 * HH
TRASH = 128
ROWSP = ROWS + TRASH

W = 128
NSUB = 16
FP = 114688
NP = 3 * FP
NW = NP // W
NWT = NW // NSUB
SLICE = ROWS // NSUB
ZCHUNK = 4096

FT = 2048
POST_T = 2048

ROLE_A = (0, 1, 2)
ROLE_B = (3, 4)


def _prep_body(vt_ref, nt_ref, a_ref, b_ref, idx_ref, zc_ref,
               nx_ref, ny_ref, nz_ref):
    t = pl.program_id(0)
    bb = pl.program_id(1)
    v = pl.program_id(2)
    posf = t * FT + lax.broadcasted_iota(jnp.int32, (1, FT), 1)
    valid = posf < F

    def row(ref, c):
        r = (v * 3 + c) * B + bb
        return ref[pl.ds(r, 1), :]

    x = row(vt_ref, 0)
    y = row(vt_ref, 1)
    z = row(vt_ref, 2)
    a0, a1, a2 = a_ref[0], a_ref[1], a_ref[2]
    b0, b1, b2 = b_ref[0], b_ref[1], b_ref[2]

    px = jnp.clip(jnp.round(x * a0 + b0), 0.0, float(H - 1)).astype(jnp.int32)
    py = jnp.clip(jnp.round(y * a1 + b1), 0.0, float(H - 1)).astype(jnp.int32)
    zv = z * a2 + b2

    nx = row(nt_ref, 0)
    ny = row(nt_ref, 1)
    nz = row(nt_ref, 2)
    w_front = (nz < 0.0).astype(jnp.int32)

    rowi = w_front * HH + py * H + px
    trash = ROWS + (posf & (TRASH - 1))
    shp = (1, FT // W, W)
    idx_ref[...] = jnp.where(valid, rowi, trash).reshape(shp)
    zero = jnp.zeros_like(zv)
    zc_ref[...] = jnp.where(valid, zv, zero).reshape(shp)
    nx_ref[...] = jnp.where(valid, nx, zero).reshape(shp)
    ny_ref[...] = jnp.where(valid, ny, zero).reshape(shp)
    nz_ref[...] = jnp.where(valid, nz, zero).reshape(shp)


def _prep(vt, nt, av, bv):
    fsteps = FP // FT
    wblk = FT // W
    grid = (fsteps, B, 3)
    last_in = (F - 1) // FT
    io_spec = pl.BlockSpec((9 * B, FT),
                           lambda t, b, v: (0, jnp.minimum(t, last_in)))
    s_spec = pl.BlockSpec(memory_space=pltpu.SMEM)
    out_spec = pl.BlockSpec((1, wblk, W),
                            lambda t, b, v: (b, v * fsteps + t, 0))
    f32 = jnp.float32
    return pl.pallas_call(
        _prep_body,
        grid=grid,
        in_specs=[io_spec, io_spec, s_spec, s_spec],
        out_specs=[out_spec] * 5,
        out_shape=[jax.ShapeDtypeStruct((B, NW, W), jnp.int32)]
        + [jax.ShapeDtypeStruct((B, NW, W), f32)] * 4,
    )(vt, nt, av, bv)


def _sc_body(idx_h, zc_h, nx_h, ny_h, nz_h, out_h,
             acc0, acc1, acc2, zbuf, onesb, idxw, valw0, valw1, dsem):
    c = lax.axis_index("c")
    s = lax.axis_index("s")
    accs = (acc0, acc1, acc2)
    srcs = (None, zc_h, nx_h, ny_h, nz_h)

    def init_zbuf(i, carry):
        zbuf[pl.ds(i * 16, 16)] = jnp.zeros((16,), jnp.float32)
        return carry

    lax.fori_loop(0, ZCHUNK // 16, init_zbuf, 0)
    for i in range(W // 16):
        onesb[pl.ds(i * 16, 16)] = jnp.ones((16,), jnp.float32)

    def do_batch(b, chans):
        nch = len(chans)

        def zero_step(i, carry):
            for k in range(nch):
                pltpu.sync_copy(zbuf, accs[k].at[pl.ds(s * SLICE + i * ZCHUNK,
                                                       ZCHUNK)])
            return carry

        lax.fori_loop(0, SLICE // ZCHUNK, zero_step, 0)
        plsc.subcore_barrier()

        w0 = s * NWT

        def stage_and_scatter(wbase, nw):
            pltpu.sync_copy(idx_h.at[b, pl.ds(wbase, nw)],
                            idxw.at[pl.ds(0, nw)])
            nxt = [valw0, valw1]
            vbufs = []
            for k in range(nch):
                if srcs[chans[k]] is None:
                    vbufs.append(None)
                else:
                    vbufs.append(nxt.pop(0))
            for k in range(nch):
                if vbufs[k] is not None:
                    pltpu.sync_copy(srcs[chans[k]].at[b, pl.ds(wbase, nw)],
                                    vbufs[k].at[pl.ds(0, nw)])

            descs = []
            for j in range(nw):
                irow = idxw.at[j]
                for k in range(nch):
                    vsrc = onesb if vbufs[k] is None else vbufs[k].at[j]
                    descs.append(
                        pltpu.async_copy(vsrc, accs[k].at[irow], dsem,
                                         add=True))
            for d in descs:
                d.wait()

        def round_body(r, carry):
            stage_and_scatter(w0 + r * 16, 16)
            return carry

        lax.fori_loop(0, NWT // 16, round_body, 0)
        stage_and_scatter(w0 + (NWT // 16) * 16, NWT % 16)

        plsc.subcore_barrier()
        for k in range(nch):
            base = (chans[k] * B + b) * ROWS
            pltpu.sync_copy(accs[k].at[pl.ds(s * SLICE, SLICE)],
                            out_h.at[pl.ds(base + s * SLICE, SLICE)])

    for b in range(B):
        a_core = b % 2

        @pl.when(c == a_core)
        def _():
            do_batch(b, ROLE_A)

        @pl.when(c != a_core)
        def _():
            do_batch(b, ROLE_B)


def _scatter(idx3, zc3, nx3, ny3, nz3):
    f32 = jnp.float32
    return pl.kernel(
        _sc_body,
        out_type=jax.ShapeDtypeStruct((5 * B * ROWS,), f32),
        mesh=plsc.VectorSubcoreMesh(core_axis_name="c", subcore_axis_name="s"),
        scratch_types=[
            pltpu.VMEM_SHARED((ROWSP,), f32),
            pltpu.VMEM_SHARED((ROWSP,), f32),
            pltpu.VMEM_SHARED((ROWSP,), f32),
            pltpu.VMEM((ZCHUNK,), f32),
            pltpu.VMEM((W,), f32),
            pltpu.VMEM((16, W), jnp.int32),
            pltpu.VMEM((16, W), f32),
            pltpu.VMEM((16, W), f32),
            pltpu.SemaphoreType.DMA,
        ],
    )(idx3, zc3, nx3, ny3, nz3)


def _post_body(bk0, bk1, bk2, bk3, bk4, fr0, fr1, fr2, fr3, fr4,
               res_ref, dF_ref, dB_ref, fx_ref, fy_ref, fz_ref,
               bx_ref, by_ref, bz_ref):
    shp = (1, 1, POST_T)

    def rd(ref):
        return ref[0, 0, :].reshape(shp)

    cB = rd(bk0)
    cF = rd(fr0)
    res_ref[...] = (cF + cB).astype(jnp.int32)
    dF_ref[...] = rd(fr1) / jnp.maximum(cF, 1.0)
    dB_ref[...] = rd(bk1) / jnp.maximum(cB, 1.0)

    fx, fy, fz = rd(fr2), rd(fr3), rd(fr4)
    tF = jnp.sqrt(fx * fx + fy * fy + fz * fz)
    tF = jnp.where(tF == 0.0, 1.0, tF)
    fx_ref[...] = fx / tF
    fy_ref[...] = fy / tF
    fz_ref[...] = fz / tF

    bx, by, bz = rd(bk2), rd(bk3), rd(bk4)
    tB = jnp.sqrt(bx * bx + by * by + bz * bz)
    tB = jnp.where(tB == 0.0, 1.0, tB)
    bx_ref[...] = bx / tB
    by_ref[...] = by / tB
    bz_ref[...] = bz / tB


def _post(acc):
    nt = HH // POST_T
    nrt = ROWS // POST_T
    grid = (B, nt)
    acc3 = acc.reshape(5 * B * nrt, 1, POST_T)

    def bk(ci):
        return pl.BlockSpec((1, 1, POST_T),
                            lambda b, t, ci=ci: ((ci * B + b) * nrt + t, 0, 0))

    def fr(ci):
        return pl.BlockSpec(
            (1, 1, POST_T),
            lambda b, t, ci=ci: ((ci * B + b) * nrt + nt + t, 0, 0))

    out_spec = pl.BlockSpec((1, 1, POST_T), lambda b, t: (b * nt + t, 0, 0))
    f32 = jnp.float32
    return pl.pallas_call(
        _post_body,
        grid=grid,
        in_specs=[bk(0), bk(1), bk(2), bk(3), bk(4),
                  fr(0), fr(1), fr(2), fr(3), fr(4)],
        out_specs=[out_spec] * 9,
        out_shape=[jax.ShapeDtypeStruct((B * nt, 1, POST_T), jnp.int32)]
        + [jax.ShapeDtypeStruct((B * nt, 1, POST_T), f32)] * 8,
    )(acc3, acc3, acc3, acc3, acc3, acc3, acc3, acc3, acc3, acc3)


def kernel(v_tensor, vn_tensor, C, a, b):
    del C
    vt = jnp.transpose(v_tensor, (2, 3, 0, 1)).reshape(9 * B, F)
    nt = jnp.transpose(vn_tensor, (2, 3, 0, 1)).reshape(9 * B, F)
    av = a.reshape(3)
    bv = b.reshape(3)

    idx, zc, nx, ny, nz = _prep(vt, nt, av, bv)
    acc = _scatter(idx, zc, nx, ny, nz).reshape(5, B, ROWS)
    res, dF, dB, fx, fy, fz, bx, by, bz = _post(acc)

    cnt = B * HH
    res = res.reshape(cnt)
    dF = dF.reshape(cnt)
    dB = dB.reshape(cnt)
    nF = jnp.stack([fx, fy, fz], axis=-1).reshape(cnt, 3)
    nB = jnp.stack([bx, by, bz], axis=-1).reshape(cnt, 3)
    return (res, dF, dB, nF, nB)
